# baseline (speedup 1.0000x reference)
import numpy as np
import jax
import jax.numpy as jnp
from jax.experimental import pallas as pl
from jax.experimental.pallas import tpu as pltpu

MAX_R = 32
S = 2 * MAX_R + 1
S2 = S // 2 + 1
NGRID = S * S * S2
NCOL = NGRID * 2
SLAB = S * S2 * 2
B = 64
SB = 128
NSHELL = MAX_R + 2


def _build_col_shell():
    ax = np.arange(-MAX_R, MAX_R + 1, dtype=np.float64)
    half = np.arange(0, MAX_R + 1, dtype=np.float64)
    zz, yy, xx = np.meshgrid(ax, ax, half, indexing="ij")
    radius = np.sqrt(zz ** 2 + yy ** 2 + xx ** 2)
    mask = radius <= MAX_R
    shell = np.where(mask, np.round(radius), MAX_R + 1).astype(np.int32)
    col_shell = np.repeat(shell.reshape(-1), 2)
    return col_shell


_COL_SHELL_NP = _build_col_shell()
_Z_NC = MAX_R
_NC_LOC0 = (MAX_R * S2 + 0) * 2


def _decoder_kernel(sb_ref, pp_ref, sf_ref, shell_ref, w_ref, out_ref):
    pp1 = pp_ref[:, 1:2]
    t = pp_ref[:, 2:3] - 5.0
    bfac = jnp.where(t > 0, t, jnp.exp(t) - 1.0) + 1.0
    s_iota = jax.lax.broadcasted_iota(jnp.int32, (B, NSHELL), 1
                                      ).astype(jnp.float32)
    table = pp1 * jnp.exp(-bfac * s_iota) * sf_ref[0][None, :]

    shells = shell_ref[0, 0]
    onehot = (shells[None, :] ==
              jax.lax.broadcasted_iota(jnp.int32, (NSHELL, SLAB), 0)
              ).astype(jnp.float32)
    scale_cols = jnp.dot(table, onehot,
                         preferred_element_type=jnp.float32)

    x = jnp.dot(sb_ref[:, :], w_ref[:, 0, 0, :],
                preferred_element_type=jnp.float32)
    out = x * scale_cols

    @pl.when(pl.program_id(0) == _Z_NC)
    def _fix_dc():
        col = jax.lax.broadcasted_iota(jnp.int32, (1, SLAB), 1)
        val0 = pp_ref[:, 0:1] * table[:, 0:1]
        nonlocal_out = jnp.where(col == _NC_LOC0, val0, out)
        nonlocal_out = jnp.where(col == _NC_LOC0 + 1, 0.0, nonlocal_out)
        _store(nonlocal_out, out_ref)

    @pl.when(pl.program_id(0) != _Z_NC)
    def _plain():
        _store(out, out_ref)


def _store(val, out_ref):
    x3 = val.reshape(B, S, S2 * 2)
    xt = jnp.transpose(x3, (0, 2, 1))
    xp = jnp.pad(xt, ((0, 0), (0, 0), (0, 128 - S)))
    out_ref[:, 0, :, :, :] = xp.reshape(B, S2, 2, 128)


@jax.jit
def kernel(sb_input, pp_input, W, spectral_factor):
    shell3 = jnp.asarray(_COL_SHELL_NP).reshape(S, 1, SLAB)
    sf_ext = jnp.concatenate(
        [spectral_factor, jnp.zeros((1,), jnp.float32)]).reshape(1, NSHELL)
    W4 = W.reshape(SB, S, 1, SLAB)
    P = pl.pallas_call(
        _decoder_kernel,
        grid=(S,),
        in_specs=[
            pl.BlockSpec((B, SB), lambda i: (0, 0)),
            pl.BlockSpec((B, 3), lambda i: (0, 0)),
            pl.BlockSpec((1, NSHELL), lambda i: (0, 0)),
            pl.BlockSpec((1, 1, SLAB), lambda i: (i, 0, 0)),
            pl.BlockSpec((SB, 1, 1, SLAB), lambda i: (0, i, 0, 0)),
        ],
        out_specs=pl.BlockSpec((B, 1, S2, 2, 128), lambda i: (0, i, 0, 0, 0)),
        out_shape=jax.ShapeDtypeStruct((B, S, S2, 2, 128), jnp.float32),
        compiler_params=pltpu.CompilerParams(
            dimension_semantics=("parallel",)),
    )(sb_input, pp_input, sf_ext, shell3, W4)
    return jnp.transpose(P, (0, 1, 4, 2, 3))[:, :, :S]


# speedup vs baseline: 1.0058x; 1.0058x over previous
import numpy as np
import jax
import jax.numpy as jnp
from jax.experimental import pallas as pl
from jax.experimental.pallas import tpu as pltpu

MAX_R = 32
S = 2 * MAX_R + 1
S2 = S // 2 + 1
NGRID = S * S * S2
NCOL = NGRID * 2
SLAB = S * S2 * 2
B = 64
SB = 128
NSHELL = MAX_R + 2


def _build_col_shell():
    ax = np.arange(-MAX_R, MAX_R + 1, dtype=np.float64)
    half = np.arange(0, MAX_R + 1, dtype=np.float64)
    zz, yy, xx = np.meshgrid(ax, ax, half, indexing="ij")
    radius = np.sqrt(zz ** 2 + yy ** 2 + xx ** 2)
    mask = radius <= MAX_R
    shell = np.where(mask, np.round(radius), MAX_R + 1).astype(np.int32)
    col_shell = np.repeat(shell.reshape(-1), 2)
    return col_shell


_COL_SHELL_NP = _build_col_shell()
_Z_NC = MAX_R
_NC_LOC0 = (MAX_R * S2 + 0) * 2


def _decoder_kernel(sb_ref, pp_ref, sf_ref, shell_ref, w_ref, out_ref):
    pp1 = pp_ref[:, 1:2]
    t = pp_ref[:, 2:3] - 5.0
    bfac = jnp.where(t > 0, t, jnp.exp(t) - 1.0) + 1.0
    s_iota = jax.lax.broadcasted_iota(jnp.int32, (B, NSHELL), 1
                                      ).astype(jnp.float32)
    table = pp1 * jnp.exp(-bfac * s_iota) * sf_ref[0][None, :]

    shells = shell_ref[0, 0]
    onehot = (shells[None, :] ==
              jax.lax.broadcasted_iota(jnp.int32, (NSHELL, SLAB), 0)
              ).astype(jnp.float32)
    scale_cols = jnp.dot(table, onehot,
                         preferred_element_type=jnp.float32)

    x = jnp.dot(sb_ref[:, :], w_ref[:, 0, 0, :],
                preferred_element_type=jnp.float32)
    out = x * scale_cols

    @pl.when(pl.program_id(0) == _Z_NC)
    def _fix_dc():
        col = jax.lax.broadcasted_iota(jnp.int32, (1, SLAB), 1)
        val0 = pp_ref[:, 0:1] * table[:, 0:1]
        nonlocal_out = jnp.where(col == _NC_LOC0, val0, out)
        nonlocal_out = jnp.where(col == _NC_LOC0 + 1, 0.0, nonlocal_out)
        _store(nonlocal_out, out_ref)

    @pl.when(pl.program_id(0) != _Z_NC)
    def _plain():
        _store(out, out_ref)


def _store(val, out_ref):
    x3 = val.reshape(B, S, S2 * 2)
    xt = jnp.transpose(x3, (0, 2, 1))
    out_ref[:, 0, :, :, :S] = xt.reshape(B, S2, 2, S)


@jax.jit
def kernel(sb_input, pp_input, W, spectral_factor):
    shell3 = jnp.asarray(_COL_SHELL_NP).reshape(S, 1, SLAB)
    sf_ext = jnp.concatenate(
        [spectral_factor, jnp.zeros((1,), jnp.float32)]).reshape(1, NSHELL)
    W4 = W.reshape(SB, S, 1, SLAB)
    P = pl.pallas_call(
        _decoder_kernel,
        grid=(S,),
        in_specs=[
            pl.BlockSpec((B, SB), lambda i: (0, 0)),
            pl.BlockSpec((B, 3), lambda i: (0, 0)),
            pl.BlockSpec((1, NSHELL), lambda i: (0, 0)),
            pl.BlockSpec((1, 1, SLAB), lambda i: (i, 0, 0)),
            pl.BlockSpec((SB, 1, 1, SLAB), lambda i: (0, i, 0, 0)),
        ],
        out_specs=pl.BlockSpec((B, 1, S2, 2, 128), lambda i: (0, i, 0, 0, 0)),
        out_shape=jax.ShapeDtypeStruct((B, S, S2, 2, 128), jnp.float32),
        compiler_params=pltpu.CompilerParams(
            dimension_semantics=("parallel",)),
    )(sb_input, pp_input, sf_ext, shell3, W4)
    return jnp.transpose(P, (0, 1, 4, 2, 3))[:, :, :S]


# speedup vs baseline: 1.0077x; 1.0019x over previous
import numpy as np
import jax
import jax.numpy as jnp
from jax.experimental import pallas as pl
from jax.experimental.pallas import tpu as pltpu

MAX_R = 32
S = 2 * MAX_R + 1
S2 = S // 2 + 1
NGRID = S * S * S2
NCOL = NGRID * 2
SLAB = S * S2 * 2
B = 64
SB = 128
NSHELL = MAX_R + 2


def _build_col_shell():
    ax = np.arange(-MAX_R, MAX_R + 1, dtype=np.float64)
    half = np.arange(0, MAX_R + 1, dtype=np.float64)
    zz, yy, xx = np.meshgrid(ax, ax, half, indexing="ij")
    radius = np.sqrt(zz ** 2 + yy ** 2 + xx ** 2)
    mask = radius <= MAX_R
    shell = np.where(mask, np.round(radius), MAX_R + 1).astype(np.int32)
    col_shell = np.repeat(shell.reshape(-1), 2)
    return col_shell


_COL_SHELL_NP = _build_col_shell()
_Z_NC = MAX_R
_NC_LOC0 = (MAX_R * S2 + 0) * 2


def _decoder_kernel(sb_ref, pp_ref, sf_ref, shell_ref, w_ref, out_ref):
    pp1 = pp_ref[:, 1:2]
    t = pp_ref[:, 2:3] - 5.0
    bfac = jnp.where(t > 0, t, jnp.exp(t) - 1.0) + 1.0
    s_iota = jax.lax.broadcasted_iota(jnp.int32, (B, NSHELL), 1
                                      ).astype(jnp.float32)
    table = pp1 * jnp.exp(-bfac * s_iota) * sf_ref[0][None, :]

    shells = shell_ref[0, 0]
    onehot = (shells[None, :] ==
              jax.lax.broadcasted_iota(jnp.int32, (NSHELL, SLAB), 0)
              ).astype(jnp.float32)
    scale_cols = jnp.dot(table, onehot,
                         preferred_element_type=jnp.float32)

    x = jnp.dot(sb_ref[:, :], w_ref[:, 0, 0, :],
                preferred_element_type=jnp.float32)
    out = x * scale_cols

    @pl.when(pl.program_id(0) == _Z_NC)
    def _fix_dc():
        col = jax.lax.broadcasted_iota(jnp.int32, (1, SLAB), 1)
        val0 = pp_ref[:, 0:1] * table[:, 0:1]
        nonlocal_out = jnp.where(col == _NC_LOC0, val0, out)
        nonlocal_out = jnp.where(col == _NC_LOC0 + 1, 0.0, nonlocal_out)
        _store(nonlocal_out, out_ref)

    @pl.when(pl.program_id(0) != _Z_NC)
    def _plain():
        _store(out, out_ref)


def _store(val, out_ref):
    x3 = val.reshape(B, S, S2 * 2)
    xt = jnp.transpose(x3, (0, 2, 1))
    out_ref[:, 0, :, :, :S] = xt.reshape(B, S2, 2, S)


@jax.jit
def kernel(sb_input, pp_input, W, spectral_factor):
    shell3 = jnp.asarray(_COL_SHELL_NP).reshape(S, 1, SLAB)
    sf_ext = jnp.concatenate(
        [spectral_factor, jnp.zeros((1,), jnp.float32)]).reshape(1, NSHELL)
    W4 = W.reshape(SB, S, 1, SLAB)
    P = pl.pallas_call(
        _decoder_kernel,
        grid=(S,),
        in_specs=[
            pl.BlockSpec((B, SB), lambda i: (0, 0)),
            pl.BlockSpec((B, 3), lambda i: (0, 0)),
            pl.BlockSpec((1, NSHELL), lambda i: (0, 0)),
            pl.BlockSpec((1, 1, SLAB), lambda i: (i, 0, 0)),
            pl.BlockSpec((SB, 1, 1, SLAB), lambda i: (0, i, 0, 0)),
        ],
        out_specs=pl.BlockSpec((B, 1, S2, 2, 128), lambda i: (0, i, 0, 0, 0)),
        out_shape=jax.ShapeDtypeStruct((B, S, S2, 2, 128), jnp.float32),
        compiler_params=pltpu.CompilerParams(
            dimension_semantics=("arbitrary",)),
    )(sb_input, pp_input, sf_ext, shell3, W4)
    return jnp.transpose(P, (0, 1, 4, 2, 3))[:, :, :S]


# speedup vs baseline: 1.0637x; 1.0556x over previous
import numpy as np
import jax
import jax.numpy as jnp
from jax.experimental import pallas as pl
from jax.experimental.pallas import tpu as pltpu

MAX_R = 32
S = 2 * MAX_R + 1
S2 = S // 2 + 1
NGRID = S * S * S2
NCOL = NGRID * 2
B = 64
SB = 128
NSHELL = MAX_R + 2
TILE = 16384


def _build_col_shell():
    ax = np.arange(-MAX_R, MAX_R + 1, dtype=np.float64)
    half = np.arange(0, MAX_R + 1, dtype=np.float64)
    zz, yy, xx = np.meshgrid(ax, ax, half, indexing="ij")
    radius = np.sqrt(zz ** 2 + yy ** 2 + xx ** 2)
    mask = radius <= MAX_R
    shell = np.where(mask, np.round(radius), MAX_R + 1).astype(np.int32)
    shell_flat = shell.reshape(-1)
    g_nc = int(np.nonzero((shell_flat == 0) & mask.reshape(-1))[0][0])
    col_shell = np.repeat(shell_flat, 2)
    return col_shell, g_nc


_COL_SHELL_NP, _G_NC = _build_col_shell()
_NC_COL0 = 2 * _G_NC
_NC_TILE = _NC_COL0 // TILE
_NC_LOC0 = _NC_COL0 - _NC_TILE * TILE


def _decoder_kernel(ppT_ref, sfT_ref, shell_ref, sb_ref, w_ref, out_ref):
    pp1 = ppT_ref[1:2, :]
    t = ppT_ref[2:3, :] - 5.0
    bfac = jnp.where(t > 0, t, jnp.exp(t) - 1.0) + 1.0
    s_iota = jax.lax.broadcasted_iota(jnp.int32, (NSHELL, B), 0
                                      ).astype(jnp.float32)
    table = pp1 * jnp.exp(-bfac * s_iota) * sfT_ref[:, :]

    shells = shell_ref[:, :]
    onehot = (shells ==
              jax.lax.broadcasted_iota(jnp.int32, (TILE, NSHELL), 1)
              ).astype(jnp.float32)
    scale_rows = jnp.dot(onehot, table,
                         preferred_element_type=jnp.float32)

    x = jax.lax.dot_general(
        w_ref[:, :], sb_ref[:, :],
        dimension_numbers=(((0,), (1,)), ((), ())),
        preferred_element_type=jnp.float32)
    out = x * scale_rows

    @pl.when(pl.program_id(0) == _NC_TILE)
    def _fix_dc():
        row = jax.lax.broadcasted_iota(jnp.int32, (TILE, 1), 0)
        val0 = ppT_ref[0:1, :] * table[0:1, :]
        fixed = jnp.where(row == _NC_LOC0, val0, out)
        fixed = jnp.where(row == _NC_LOC0 + 1, 0.0, fixed)
        out_ref[:, :] = fixed

    @pl.when(pl.program_id(0) != _NC_TILE)
    def _plain():
        out_ref[:, :] = out


@jax.jit
def kernel(sb_input, pp_input, W, spectral_factor):
    shell_col = jnp.asarray(_COL_SHELL_NP).reshape(NCOL, 1)
    sfT = jnp.concatenate(
        [spectral_factor, jnp.zeros((1,), jnp.float32)]).reshape(NSHELL, 1)
    ppT = pp_input.T
    grid = (NCOL + TILE - 1) // TILE
    outT = pl.pallas_call(
        _decoder_kernel,
        grid=(grid,),
        in_specs=[
            pl.BlockSpec((3, B), lambda i: (0, 0)),
            pl.BlockSpec((NSHELL, 1), lambda i: (0, 0)),
            pl.BlockSpec((TILE, 1), lambda i: (i, 0)),
            pl.BlockSpec((B, SB), lambda i: (0, 0)),
            pl.BlockSpec((SB, TILE), lambda i: (0, i)),
        ],
        out_specs=pl.BlockSpec((TILE, B), lambda i: (i, 0)),
        out_shape=jax.ShapeDtypeStruct((NCOL, B), jnp.float32),
        compiler_params=pltpu.CompilerParams(
            dimension_semantics=("parallel",)),
    )(ppT, sfT, shell_col, sb_input, W)
    return outT.T.reshape(B, S, S, S2, 2)
